# baseline (speedup 1.0000x reference)
import functools
import math

import jax
import jax.numpy as jnp
from jax import lax
from jax.experimental import pallas as pl
from jax.experimental.pallas import tpu as pltpu
from jax.experimental.pallas import tpu_sc as plsc

VOCAB = 1_000_000
D = 64
B_ROWS = 4096
B_COLS = 200
B_TOTAL = B_ROWS * B_COLS

NC = 2
NS = 16
NW = NC * NS
PER_W = B_TOTAL // NW
GW = 128
CHUNK = 256
NROW = PER_W // GW
NG = PER_W // CHUNK
SCALE = math.sqrt(D)

_mesh = plsc.VectorSubcoreMesh(core_axis_name="c", subcore_axis_name="s")


@functools.partial(
    pl.kernel,
    out_type=jax.ShapeDtypeStruct((B_TOTAL, D), jnp.float32),
    mesh=_mesh,
    compiler_params=pltpu.CompilerParams(use_tc_tiling_on_sc=False),
    scratch_types=[
        pltpu.VMEM((NROW, GW), jnp.int32),
        pltpu.VMEM((CHUNK, D), jnp.float32),
        pltpu.VMEM((CHUNK, D), jnp.float32),
        pltpu.VMEM((CHUNK, D), jnp.float32),
        pltpu.VMEM((CHUNK, D), jnp.float32),
        pltpu.SemaphoreType.DMA,
        pltpu.SemaphoreType.DMA,
        pltpu.SemaphoreType.DMA,
        pltpu.SemaphoreType.DMA,
    ],
)
def _emb_kernel(tokens_hbm, table_hbm, out_hbm,
                idx_v, r0, r1, w0, w1, sg0, sg1, sw0, sw1):
    wid = lax.axis_index("s") * NC + lax.axis_index("c")
    base = wid * PER_W
    pltpu.sync_copy(tokens_hbm.at[wid], idx_v)

    def g_start(c, rbuf, sem):
        pltpu.async_copy(table_hbm.at[idx_v.at[2 * c]],
                         rbuf.at[pl.ds(0, GW)], sem)
        pltpu.async_copy(table_hbm.at[idx_v.at[2 * c + 1]],
                         rbuf.at[pl.ds(GW, GW)], sem)

    def g_wait(rbuf, sem):
        pltpu.make_async_copy(table_hbm.at[idx_v.at[0]],
                              rbuf.at[pl.ds(0, GW)], sem).wait()
        pltpu.make_async_copy(table_hbm.at[idx_v.at[0]],
                              rbuf.at[pl.ds(GW, GW)], sem).wait()

    def w_start(c, wbuf, sem):
        pltpu.async_copy(wbuf, out_hbm.at[pl.ds(base + c * CHUNK, CHUNK)], sem)

    def w_wait(wbuf, sem):
        pltpu.make_async_copy(wbuf, out_hbm.at[pl.ds(base, CHUNK)], sem).wait()

    def scale_chunk(rbuf, wbuf):
        def srow(r, carry):
            for k in range(D // 16):
                sl = pl.ds(k * 16, 16)
                wbuf[r, sl] = rbuf[r, sl] * SCALE
            return carry
        lax.fori_loop(0, CHUNK, srow, 0, unroll=4)

    bufs = ((r0, w0, sg0, sw0), (r1, w1, sg1, sw1))

    g_start(0, r0, sg0)
    g_start(1, r1, sg1)
    for p in range(2):
        rb, wb, sg, sw = bufs[p]
        g_wait(rb, sg)
        scale_chunk(rb, wb)
        w_start(p, wb, sw)
        g_start(p + 2, rb, sg)

    def step(i, carry):
        for p in range(2):
            c = 2 * i + p
            rb, wb, sg, sw = bufs[p]
            g_wait(rb, sg)
            w_wait(wb, sw)
            scale_chunk(rb, wb)
            w_start(c, wb, sw)
            g_start(c + 2, rb, sg)
        return carry

    lax.fori_loop(1, NG // 2 - 1, step, 0)

    for p in range(2):
        c = NG - 2 + p
        rb, wb, sg, sw = bufs[p]
        g_wait(rb, sg)
        w_wait(wb, sw)
        scale_chunk(rb, wb)
        w_start(c, wb, sw)
    for p in range(2):
        rb, wb, sg, sw = bufs[p]
        w_wait(wb, sw)


def kernel(tokens, table):
    idx = tokens.reshape(NW, NROW, GW)
    out = _emb_kernel(idx, table)
    return out.reshape(B_ROWS, B_COLS, D)
